# baseline (speedup 1.0000x reference)
import functools

import jax
import jax.numpy as jnp
from jax import lax
from jax.experimental import pallas as pl
from jax.experimental.pallas import tpu as pltpu
from jax.experimental.pallas import tpu_sc as plsc

_B = 16384
_V = 128
_L = 16

_info = plsc.get_sparse_core_info()
_NW = _info.num_subcores
_BPW = _B // _NW

_GATHER_DNUMS = lax.GatherDimensionNumbers(
    offset_dims=(), collapsed_slice_dims=(0,), start_index_map=(0,)
)


def _vgather16(vec16, idx16):
    return lax.gather(
        vec16,
        idx16[:, None],
        _GATHER_DNUMS,
        slice_sizes=(1,),
        mode=lax.GatherScatterMode.PROMISE_IN_BOUNDS,
    )


def _lookup_body(labels_hbm, table_hbm, out_hbm, idx_v, tab_v, out_v, sem):
    wid = lax.axis_index("s")
    base = wid * _BPW
    c_tab = pltpu.async_copy(table_hbm, tab_v, sem)
    c_idx = pltpu.async_copy(labels_hbm.at[pl.ds(base, _BPW)], idx_v, sem)
    c_tab.wait()
    c_idx.wait()
    tabs = [tab_v[pl.ds(k * _L, _L)] for k in range(_V // _L)]

    def chunk(i, carry):
        idx = idx_v[pl.ds(i * _L, _L)]
        lo = lax.bitwise_and(idx, _L - 1)
        hi = lax.shift_right_logical(idx, 4)
        res = _vgather16(tabs[0], lo)
        for k in range(1, _V // _L):
            res = jnp.where(hi == k, _vgather16(tabs[k], lo), res)
        out_v[pl.ds(i * _L, _L)] = res
        return carry

    lax.fori_loop(0, _BPW // _L, chunk, 0, unroll=8)
    pltpu.sync_copy(out_v, out_hbm.at[pl.ds(base, _BPW)])


_mesh = plsc.VectorSubcoreMesh(
    core_axis_name="c", subcore_axis_name="s", num_cores=1
)

_lookup = functools.partial(
    pl.kernel,
    mesh=_mesh,
    out_type=jax.ShapeDtypeStruct((_B,), jnp.int32),
    scratch_types=[
        pltpu.VMEM((_BPW,), jnp.int32),
        pltpu.VMEM((_V,), jnp.int32),
        pltpu.VMEM((_BPW,), jnp.int32),
        pltpu.SemaphoreType.DMA,
    ],
)(_lookup_body)


@jax.jit
def kernel(tactic_labels, tactic_index_to_numargs):
    labels = tactic_labels.astype(jnp.int32)
    table = tactic_index_to_numargs.astype(jnp.int32)
    return _lookup(labels, table)


# speedup vs baseline: 1.0239x; 1.0239x over previous
import functools

import jax
import jax.numpy as jnp
from jax import lax
from jax.experimental import pallas as pl
from jax.experimental.pallas import tpu as pltpu
from jax.experimental.pallas import tpu_sc as plsc

_B = 16384
_V = 128
_L = 16

_info = plsc.get_sparse_core_info()
_NW = _info.num_subcores
_BPW = _B // _NW

_GATHER_DNUMS = lax.GatherDimensionNumbers(
    offset_dims=(), collapsed_slice_dims=(0,), start_index_map=(0,)
)


def _vgather16(vec16, idx16):
    return lax.gather(
        vec16,
        idx16[:, None],
        _GATHER_DNUMS,
        slice_sizes=(1,),
        mode=lax.GatherScatterMode.PROMISE_IN_BOUNDS,
    )


def _lookup_body(labels_hbm, table_hbm, out_hbm, idx_v, tab_v, out_v, sem):
    wid = lax.axis_index("s")
    base = wid * _BPW
    c_tab = pltpu.async_copy(table_hbm, tab_v, sem)
    c_idx = pltpu.async_copy(labels_hbm.at[pl.ds(base, _BPW)], idx_v, sem)
    c_tab.wait()
    c_idx.wait()
    tabs = [tab_v[pl.ds(k * _L, _L)] for k in range(_V // _L)]

    def chunk(i, carry):
        idx = idx_v[pl.ds(i * _L, _L)]
        lo = lax.bitwise_and(idx, _L - 1)
        hi = lax.shift_right_logical(idx, 4)
        res = _vgather16(tabs[0], lo)
        for k in range(1, _V // _L):
            res = jnp.where(hi == k, _vgather16(tabs[k], lo), res)
        out_v[pl.ds(i * _L, _L)] = res
        return carry

    lax.fori_loop(0, _BPW // _L, chunk, 0, unroll=2)
    pltpu.sync_copy(out_v, out_hbm.at[pl.ds(base, _BPW)])


_mesh = plsc.VectorSubcoreMesh(
    core_axis_name="c", subcore_axis_name="s", num_cores=1
)

_lookup = functools.partial(
    pl.kernel,
    mesh=_mesh,
    out_type=jax.ShapeDtypeStruct((_B,), jnp.int32),
    scratch_types=[
        pltpu.VMEM((_BPW,), jnp.int32),
        pltpu.VMEM((_V,), jnp.int32),
        pltpu.VMEM((_BPW,), jnp.int32),
        pltpu.SemaphoreType.DMA,
    ],
)(_lookup_body)


@jax.jit
def kernel(tactic_labels, tactic_index_to_numargs):
    labels = tactic_labels.astype(jnp.int32)
    table = tactic_index_to_numargs.astype(jnp.int32)
    return _lookup(labels, table)


# speedup vs baseline: 1.0436x; 1.0193x over previous
import functools

import jax
import jax.numpy as jnp
from jax import lax
from jax.experimental import pallas as pl
from jax.experimental.pallas import tpu as pltpu
from jax.experimental.pallas import tpu_sc as plsc

_B = 16384
_V = 128
_L = 16

_info = plsc.get_sparse_core_info()
_NW = _info.num_subcores
_BPW = _B // _NW

_GATHER_DNUMS = lax.GatherDimensionNumbers(
    offset_dims=(), collapsed_slice_dims=(0,), start_index_map=(0,)
)


def _vgather16(vec16, idx16):
    return lax.gather(
        vec16,
        idx16[:, None],
        _GATHER_DNUMS,
        slice_sizes=(1,),
        mode=lax.GatherScatterMode.PROMISE_IN_BOUNDS,
    )


def _lookup_body(labels_hbm, table_hbm, out_hbm, idx_v, tab_v, out_v, sem):
    wid = lax.axis_index("s")
    base = wid * _BPW
    c_tab = pltpu.async_copy(table_hbm, tab_v, sem)
    c_idx = pltpu.async_copy(labels_hbm.at[pl.ds(base, _BPW)], idx_v, sem)
    c_tab.wait()
    c_idx.wait()
    tabs = [tab_v[pl.ds(k * _L, _L)] for k in range(_V // _L)]

    def chunk(i, carry):
        idx = idx_v[pl.ds(i * _L, _L)]
        lo = lax.bitwise_and(idx, _L - 1)
        hi = lax.shift_right_logical(idx, 4)
        res = _vgather16(tabs[0], lo)
        for k in range(1, _V // _L):
            res = jnp.where(hi == k, _vgather16(tabs[k], lo), res)
        out_v[pl.ds(i * _L, _L)] = res
        return carry

    lax.fori_loop(0, _BPW // _L, chunk, 0, unroll=1)
    pltpu.sync_copy(out_v, out_hbm.at[pl.ds(base, _BPW)])


_mesh = plsc.VectorSubcoreMesh(
    core_axis_name="c", subcore_axis_name="s", num_cores=1
)

_lookup = functools.partial(
    pl.kernel,
    mesh=_mesh,
    out_type=jax.ShapeDtypeStruct((_B,), jnp.int32),
    scratch_types=[
        pltpu.VMEM((_BPW,), jnp.int32),
        pltpu.VMEM((_V,), jnp.int32),
        pltpu.VMEM((_BPW,), jnp.int32),
        pltpu.SemaphoreType.DMA,
    ],
)(_lookup_body)


@jax.jit
def kernel(tactic_labels, tactic_index_to_numargs):
    labels = tactic_labels.astype(jnp.int32)
    table = tactic_index_to_numargs.astype(jnp.int32)
    return _lookup(labels, table)


# speedup vs baseline: 1.0444x; 1.0008x over previous
import functools

import jax
import jax.numpy as jnp
from jax import lax
from jax.experimental import pallas as pl
from jax.experimental.pallas import tpu as pltpu
from jax.experimental.pallas import tpu_sc as plsc

_B = 16384
_V = 128
_L = 16

_info = plsc.get_sparse_core_info()
_NW = _info.num_subcores
_BPW = _B // _NW

_GATHER_DNUMS = lax.GatherDimensionNumbers(
    offset_dims=(), collapsed_slice_dims=(0,), start_index_map=(0,)
)


def _vgather16(vec16, idx16):
    return lax.gather(
        vec16,
        idx16[:, None],
        _GATHER_DNUMS,
        slice_sizes=(1,),
        mode=lax.GatherScatterMode.PROMISE_IN_BOUNDS,
    )


def _lookup_body(labels_hbm, table_hbm, out_hbm, idx_v, tab_v, out_v, sem):
    wid = lax.axis_index("s")
    base = wid * _BPW
    c_tab = pltpu.async_copy(table_hbm, tab_v, sem)
    c_idx = pltpu.async_copy(labels_hbm.at[pl.ds(base, _BPW)], idx_v, sem)
    c_tab.wait()
    c_idx.wait()
    tabs = [tab_v[pl.ds(k * _L, _L)] for k in range(_V // _L)]

    @plsc.parallel_loop(0, _BPW, step=_L)
    def chunk(i):
        idx = idx_v[pl.ds(i, _L)]
        lo = lax.bitwise_and(idx, _L - 1)
        hi = lax.shift_right_logical(idx, 4)
        res = _vgather16(tabs[0], lo)
        for k in range(1, _V // _L):
            res = jnp.where(hi == k, _vgather16(tabs[k], lo), res)
        out_v[pl.ds(i, _L)] = res
    pltpu.sync_copy(out_v, out_hbm.at[pl.ds(base, _BPW)])


_mesh = plsc.VectorSubcoreMesh(
    core_axis_name="c", subcore_axis_name="s", num_cores=1
)

_lookup = functools.partial(
    pl.kernel,
    mesh=_mesh,
    out_type=jax.ShapeDtypeStruct((_B,), jnp.int32),
    scratch_types=[
        pltpu.VMEM((_BPW,), jnp.int32),
        pltpu.VMEM((_V,), jnp.int32),
        pltpu.VMEM((_BPW,), jnp.int32),
        pltpu.SemaphoreType.DMA,
    ],
)(_lookup_body)


@jax.jit
def kernel(tactic_labels, tactic_index_to_numargs):
    labels = tactic_labels.astype(jnp.int32)
    table = tactic_index_to_numargs.astype(jnp.int32)
    return _lookup(labels, table)
